# baseline (device time: 27120 ns/iter reference)
import os

import jax
import jax.numpy as jnp
from jax import lax
from jax.experimental import pallas as pl
from jax.experimental.pallas import tpu as pltpu

_NO_COMM = os.environ.get("KERNEL_NO_COMM", "0") == "1"

N_LAYERS = 3
N_CHUNKS = 2


def _dot(a, b):
    return lax.dot_general(
        a, b, (((1,), (0,)), ((), ())), preferred_element_type=jnp.float32
    )


def kernel(x, Win0, Wout0, Win1, Wout1, Win2, Wout2):
    b, d_local = x.shape
    _, h_local = Win0.shape
    chunk = h_local // N_CHUNKS

    def body(x_ref, win0_ref, wout0_ref, win1_ref, wout1_ref, win2_ref,
             wout2_ref, out_ref,
             ysend_ref, yrecv_ref, xsend_ref, xrecv_ref,
             send_sems, recv_sems):
        my_x = lax.axis_index("x")
        my_y = lax.axis_index("y")
        y_nbr = (my_x, 1 - my_y)
        x_nbr = (1 - my_x, my_y)

        if not _NO_COMM:
            barrier_sem = pltpu.get_barrier_semaphore()
            for nbr in (y_nbr, x_nbr):
                pl.semaphore_signal(
                    barrier_sem, inc=1,
                    device_id=nbr, device_id_type=pl.DeviceIdType.MESH,
                )
            pl.semaphore_wait(barrier_sem, 2)

        wins = (win0_ref, win1_ref, win2_ref)
        wouts = (wout0_ref, wout1_ref, wout2_ref)

        def rdma(buf_send, buf_recv, sem_idx, nbr):
            return pltpu.make_async_remote_copy(
                src_ref=buf_send,
                dst_ref=buf_recv,
                send_sem=send_sems.at[sem_idx],
                recv_sem=recv_sems.at[sem_idx],
                device_id=nbr,
                device_id_type=pl.DeviceIdType.MESH,
            )

        pending_sends = []
        x_cur = x_ref[...]
        for k in range(N_LAYERS):
            win = wins[k][...]
            wout = wouts[k][...]

            rdmas_y = []
            for c in range(N_CHUNKS):
                cols = slice(c * chunk, (c + 1) * chunk)
                ysend_ref[k, c] = _dot(x_cur, win[:, cols])
                r = rdma(ysend_ref.at[k, c], yrecv_ref.at[k, c],
                         2 * N_CHUNKS * k + c, y_nbr)
                if not _NO_COMM:
                    r.start()
                rdmas_y.append(r)

            rdmas_x = []
            for c in range(N_CHUNKS):
                rows = slice(c * chunk, (c + 1) * chunk)
                if not _NO_COMM:
                    rdmas_y[c].wait_recv()
                h_c = jnp.maximum(ysend_ref[k, c] + yrecv_ref[k, c], 0.0)
                xsend_ref[k, c] = _dot(h_c, wout[rows, :])
                r = rdma(xsend_ref.at[k, c], xrecv_ref.at[k, c],
                         2 * N_CHUNKS * k + N_CHUNKS + c, x_nbr)
                if not _NO_COMM:
                    r.start()
                rdmas_x.append(r)
            pending_sends.extend(rdmas_y)
            pending_sends.extend(rdmas_x)

            if not _NO_COMM:
                for r in rdmas_x:
                    r.wait_recv()
            x_cur = (
                (xsend_ref[k, 0] + xsend_ref[k, 1])
                + (xrecv_ref[k, 0] + xrecv_ref[k, 1])
            )

        out_ref[...] = x_cur

        if not _NO_COMM:
            for r in pending_sends:
                r.wait_send()

    n_msgs = 2 * N_CHUNKS * N_LAYERS
    return pl.pallas_call(
        body,
        out_shape=jax.ShapeDtypeStruct((b, d_local), jnp.float32),
        in_specs=[pl.BlockSpec(memory_space=pltpu.VMEM)] * 7,
        out_specs=pl.BlockSpec(memory_space=pltpu.VMEM),
        scratch_shapes=[
            pltpu.VMEM((N_LAYERS, N_CHUNKS, b, chunk), jnp.float32),
            pltpu.VMEM((N_LAYERS, N_CHUNKS, b, chunk), jnp.float32),
            pltpu.VMEM((N_LAYERS, N_CHUNKS, b, d_local), jnp.float32),
            pltpu.VMEM((N_LAYERS, N_CHUNKS, b, d_local), jnp.float32),
            pltpu.SemaphoreType.DMA((n_msgs,)),
            pltpu.SemaphoreType.DMA((n_msgs,)),
        ],
        compiler_params=pltpu.CompilerParams(
            collective_id=None if _NO_COMM else 0
        ),
    )(x, Win0, Wout0, Win1, Wout1, Win2, Wout2)


# device time: 24016 ns/iter; 1.1292x vs baseline; 1.1292x over previous
import os

import jax
import jax.numpy as jnp
from jax import lax
from jax.experimental import pallas as pl
from jax.experimental.pallas import tpu as pltpu

_NO_COMM = os.environ.get("KERNEL_NO_COMM", "0") == "1"
_CHAIN_ONLY = os.environ.get("KERNEL_CHAIN_ONLY", "0") == "1"

N_LAYERS = 3
N_CHUNKS = 2


def _dot(a, b):
    return lax.dot_general(
        a, b, (((1,), (0,)), ((), ())), preferred_element_type=jnp.float32
    )


def kernel(x, Win0, Wout0, Win1, Wout1, Win2, Wout2):
    b, d_local = x.shape
    _, h_local = Win0.shape
    chunk = h_local // N_CHUNKS

    def body(x_ref, win0_ref, wout0_ref, win1_ref, wout1_ref, win2_ref,
             wout2_ref, out_ref,
             ysend_ref, yrecv_ref, xsend_ref, xrecv_ref,
             send_sems, recv_sems):
        my_x = lax.axis_index("x")
        my_y = lax.axis_index("y")
        y_nbr = (my_x, 1 - my_y)
        x_nbr = (1 - my_x, my_y)

        if not _NO_COMM:
            barrier_sem = pltpu.get_barrier_semaphore()
            for nbr in (y_nbr, x_nbr):
                pl.semaphore_signal(
                    barrier_sem, inc=1,
                    device_id=nbr, device_id_type=pl.DeviceIdType.MESH,
                )
            pl.semaphore_wait(barrier_sem, 2)

        wins = (win0_ref, win1_ref, win2_ref)
        wouts = (wout0_ref, wout1_ref, wout2_ref)

        def rdma(buf_send, buf_recv, sem_idx, nbr):
            return pltpu.make_async_remote_copy(
                src_ref=buf_send,
                dst_ref=buf_recv,
                send_sem=send_sems.at[sem_idx],
                recv_sem=recv_sems.at[sem_idx],
                device_id=nbr,
                device_id_type=pl.DeviceIdType.MESH,
            )

        if _CHAIN_ONLY:
            chain = []
            for k in range(N_LAYERS):
                ry = rdma(ysend_ref.at[k], yrecv_ref.at[k], 2 * k, y_nbr)
                ry.start()
                ry.wait_recv()
                rx = rdma(xsend_ref.at[k, 0], xrecv_ref.at[k, 0],
                          2 * k + 1, x_nbr)
                rx.start()
                rx.wait_recv()
                chain.extend([ry, rx])
            out_ref[...] = x_ref[...]
            for r in chain:
                r.wait_send()
            return

        pending_sends = []
        x_cur = x_ref[...]
        for k in range(N_LAYERS):
            win = wins[k][...]
            wout = wouts[k][...]

            rdmas_y = []
            for c in range(N_CHUNKS):
                cols = slice(c * chunk, (c + 1) * chunk)
                ysend_ref[k, c] = _dot(x_cur, win[:, cols]).astype(
                    jnp.bfloat16
                )
                r = rdma(ysend_ref.at[k, c], yrecv_ref.at[k, c],
                         2 * N_CHUNKS * k + c, y_nbr)
                if not _NO_COMM:
                    r.start()
                rdmas_y.append(r)

            rdmas_x = []
            for c in range(N_CHUNKS):
                rows = slice(c * chunk, (c + 1) * chunk)
                if not _NO_COMM:
                    rdmas_y[c].wait_recv()
                h_c = jnp.maximum(
                    ysend_ref[k, c].astype(jnp.float32)
                    + yrecv_ref[k, c].astype(jnp.float32),
                    0.0,
                )
                xsend_ref[k, c] = _dot(h_c, wout[rows, :]).astype(
                    jnp.bfloat16
                )
                r = rdma(xsend_ref.at[k, c], xrecv_ref.at[k, c],
                         2 * N_CHUNKS * k + N_CHUNKS + c, x_nbr)
                if not _NO_COMM:
                    r.start()
                rdmas_x.append(r)
            pending_sends.extend(rdmas_y)
            pending_sends.extend(rdmas_x)

            if not _NO_COMM:
                for r in rdmas_x:
                    r.wait_recv()
            x_cur = (
                (xsend_ref[k, 0].astype(jnp.float32)
                 + xsend_ref[k, 1].astype(jnp.float32))
                + (xrecv_ref[k, 0].astype(jnp.float32)
                   + xrecv_ref[k, 1].astype(jnp.float32))
            )

        out_ref[...] = x_cur

        if not _NO_COMM:
            for r in pending_sends:
                r.wait_send()

    n_msgs = 2 * N_CHUNKS * N_LAYERS
    return pl.pallas_call(
        body,
        out_shape=jax.ShapeDtypeStruct((b, d_local), jnp.float32),
        in_specs=[pl.BlockSpec(memory_space=pltpu.VMEM)] * 7,
        out_specs=pl.BlockSpec(memory_space=pltpu.VMEM),
        scratch_shapes=[
            pltpu.VMEM((N_LAYERS, N_CHUNKS, b, chunk), jnp.bfloat16),
            pltpu.VMEM((N_LAYERS, N_CHUNKS, b, chunk), jnp.bfloat16),
            pltpu.VMEM((N_LAYERS, N_CHUNKS, b, d_local), jnp.bfloat16),
            pltpu.VMEM((N_LAYERS, N_CHUNKS, b, d_local), jnp.bfloat16),
            pltpu.SemaphoreType.DMA((n_msgs,)),
            pltpu.SemaphoreType.DMA((n_msgs,)),
        ],
        compiler_params=pltpu.CompilerParams(
            collective_id=None if _NO_COMM else 0
        ),
    )(x, Win0, Wout0, Win1, Wout1, Win2, Wout2)
